# baseline (speedup 1.0000x reference)
import jax
import jax.numpy as jnp
from jax import lax
from jax.experimental import pallas as pl
from jax.experimental.pallas import tpu as pltpu
from jax.experimental.pallas import tpu_sc as plsc

B = 16384
D = 32
H1 = 256
H2 = 64

_NC = 2
_NS = 16
_NW = _NC * _NS
_BPW = B // _NW
_CHUNK = 128
_NCHUNK = _BPW // _CHUNK


def _gather_body(uid_ref, iid_ref, utab_ref, itab_ref, uout_ref, iout_ref,
                 uidx_v, iidx_v, urows_v, irows_v, sem_u, sem_i):
    wid = lax.axis_index("s") * _NC + lax.axis_index("c")
    base = wid * _BPW
    row0 = wid * _NCHUNK
    pltpu.sync_copy(uid_ref.at[pl.ds(row0, _NCHUNK)], uidx_v)
    pltpu.sync_copy(iid_ref.at[pl.ds(row0, _NCHUNK)], iidx_v)
    copies = []
    for j in range(_NCHUNK):
        copies.append(pltpu.async_copy(
            utab_ref.at[uidx_v.at[j]],
            urows_v.at[pl.ds(j * _CHUNK, _CHUNK)], sem_u))
        copies.append(pltpu.async_copy(
            itab_ref.at[iidx_v.at[j]],
            irows_v.at[pl.ds(j * _CHUNK, _CHUNK)], sem_i))
    for c in copies:
        c.wait()
    pltpu.sync_copy(urows_v, uout_ref.at[pl.ds(base, _BPW)])
    pltpu.sync_copy(irows_v, iout_ref.at[pl.ds(base, _BPW)])


def _sc_gather(uid2d, iid2d, utab, itab):
    mesh = plsc.VectorSubcoreMesh(core_axis_name="c", subcore_axis_name="s")
    f = pl.kernel(
        _gather_body,
        mesh=mesh,
        compiler_params=pltpu.CompilerParams(use_tc_tiling_on_sc=False),
        out_type=[jax.ShapeDtypeStruct((B, D), jnp.float32),
                  jax.ShapeDtypeStruct((B, D), jnp.float32)],
        scratch_types=[
            pltpu.VMEM((_NCHUNK, _CHUNK), jnp.int32),
            pltpu.VMEM((_NCHUNK, _CHUNK), jnp.int32),
            pltpu.VMEM((_BPW, D), jnp.float32),
            pltpu.VMEM((_BPW, D), jnp.float32),
            pltpu.SemaphoreType.DMA,
            pltpu.SemaphoreType.DMA,
        ],
    )
    return f(uid2d, iid2d, utab, itab)


_BM = 1024


def _mlp_body(u_ref, i_ref, w1a_ref, w1b_ref, b1_ref, w2_ref, b2_ref,
              w3t_ref, b3_ref, o_ref):
    h = jnp.dot(u_ref[...], w1a_ref[...], preferred_element_type=jnp.float32)
    h = h + jnp.dot(i_ref[...], w1b_ref[...], preferred_element_type=jnp.float32)
    h = jnp.maximum(h + b1_ref[...], 0.0)
    h = jnp.dot(h, w2_ref[...], preferred_element_type=jnp.float32)
    h = jnp.maximum(h + b2_ref[...], 0.0)
    o_ref[...] = jnp.sum(h * w3t_ref[...], axis=1, keepdims=True) + b3_ref[...]


def _tc_mlp(u_emb, i_emb, W1a, W1b, b1r, W2, b2r, W3t, b3r):
    return pl.pallas_call(
        _mlp_body,
        grid=(B // _BM,),
        in_specs=[
            pl.BlockSpec((_BM, D), lambda i: (i, 0)),
            pl.BlockSpec((_BM, D), lambda i: (i, 0)),
            pl.BlockSpec((D, H1), lambda i: (0, 0)),
            pl.BlockSpec((D, H1), lambda i: (0, 0)),
            pl.BlockSpec((1, H1), lambda i: (0, 0)),
            pl.BlockSpec((H1, H2), lambda i: (0, 0)),
            pl.BlockSpec((1, H2), lambda i: (0, 0)),
            pl.BlockSpec((1, H2), lambda i: (0, 0)),
            pl.BlockSpec((1, 1), lambda i: (0, 0)),
        ],
        out_specs=pl.BlockSpec((_BM, 1), lambda i: (i, 0)),
        out_shape=jax.ShapeDtypeStruct((B, 1), jnp.float32),
    )(u_emb, i_emb, W1a, W1b, b1r, W2, b2r, W3t, b3r)


def kernel(user_id, item_name, user_table, item_table, W1, b1, W2, b2, W3, b3):
    uid2d = user_id.reshape(B // _CHUNK, _CHUNK)
    iid2d = item_name.reshape(B // _CHUNK, _CHUNK)
    u_emb, i_emb = _sc_gather(uid2d, iid2d, user_table, item_table)
    return _tc_mlp(u_emb, i_emb,
                   W1[:D], W1[D:], b1.reshape(1, H1),
                   W2, b2.reshape(1, H2),
                   W3.reshape(1, H2), b3.reshape(1, 1))


# speedup vs baseline: 1.6495x; 1.6495x over previous
import functools

import jax
import jax.numpy as jnp
from jax import lax
from jax.experimental import pallas as pl
from jax.experimental.pallas import tpu as pltpu
from jax.experimental.pallas import tpu_sc as plsc

B = 16384
D = 32
H1 = 256
H2 = 64
V = 1000001

_NC = 2
_NS = 16
_NW = _NC * _NS
_BPW = B // _NW
_CH = 128
_NCH = _BPW // _CH

_VBLK = 4096
_PBLK = _VBLK // 4
_NBLK = (V + _VBLK - 1) // _VBLK
_PROWS = _NBLK * _PBLK


def _pack_body(ut_ref, it_ref, eye_ref, up_ref, ip_ref):
    eye = eye_ref[...]
    for src, dst in ((ut_ref, up_ref), (it_ref, ip_ref)):
        for j in range(4):
            xj = src[:, j * _PBLK:(j + 1) * _PBLK]
            dst[:, j * D:(j + 1) * D] = lax.dot_general(
                xj, eye, (((0,), (0,)), ((), ())),
                preferred_element_type=jnp.float32)


def _pack_tables(utabT, itabT, eye):
    return pl.pallas_call(
        _pack_body,
        grid=(_NBLK,),
        in_specs=[
            pl.BlockSpec((D, _VBLK), lambda i: (0, i)),
            pl.BlockSpec((D, _VBLK), lambda i: (0, i)),
            pl.BlockSpec((D, D), lambda i: (0, 0)),
        ],
        out_specs=[
            pl.BlockSpec((_PBLK, 4 * D), lambda i: (i, 0)),
            pl.BlockSpec((_PBLK, 4 * D), lambda i: (i, 0)),
        ],
        out_shape=[jax.ShapeDtypeStruct((_PROWS, 4 * D), jnp.float32),
                   jax.ShapeDtypeStruct((_PROWS, 4 * D), jnp.float32)],
    )(utabT, itabT, eye)


def _gather_body(uid_ref, iid_ref, upk_ref, ipk_ref, out_ref,
                 uidx_v, iidx_v, upr_v, ipr_v, uja_v, ija_v,
                 slot_a, slot_b, xbuf_v, sem):
    wid = lax.axis_index("s") * _NC + lax.axis_index("c")
    base = wid * _BPW
    pltpu.sync_copy(uid_ref.at[pl.ds(base, _BPW)], uidx_v)
    pltpu.sync_copy(iid_ref.at[pl.ds(base, _BPW)], iidx_v)

    for idx_v, pr_v, ja_v in ((uidx_v, upr_v, uja_v), (iidx_v, ipr_v, ija_v)):
        for g in range(_BPW // 16):
            rv = idx_v[pl.ds(16 * g, 16)]
            pr_v[pl.ds(16 * g, 16)] = ((rv >> 12) << 10) | (rv & 1023)
            ja_v[pl.ds(16 * g, 16)] = ((rv >> 10) & 3) * D

    work = [(upk_ref, upr_v, uja_v, 0), (ipk_ref, ipr_v, ija_v, D)]
    slots = (slot_a, slot_b)

    def fire(k, slot):
        pk_ref, pr_v, _, _ = work[k // _NCH]
        ch = k % _NCH
        return pltpu.async_copy(
            pk_ref.at[pr_v.at[pl.ds(ch * _CH, _CH)]], slot, sem)

    def extract(k, slot):
        _, _, ja_v, foff = work[k // _NCH]
        ch = k % _NCH

        def per_f(f, _):
            for g2 in range(_CH // 16):
                e_vec = lax.iota(jnp.int32, 16) + (16 * g2)
                ja = ja_v[pl.ds(ch * _CH + 16 * g2, 16)]
                vals = plsc.load_gather(slot, [e_vec, ja + f])
                xbuf_v[foff + f, pl.ds(ch * _CH + 16 * g2, 16)] = vals
            return 0

        lax.fori_loop(0, D, per_f, 0)

    copies = {0: fire(0, slots[0])}
    for k in range(2 * _NCH):
        if k + 1 < 2 * _NCH:
            copies[k + 1] = fire(k + 1, slots[(k + 1) % 2])
        copies[k].wait()
        extract(k, slots[k % 2])

    pltpu.sync_copy(xbuf_v, out_ref.at[:, pl.ds(base, _BPW)])


def _sc_gather(uid, iid, upk, ipk):
    mesh = plsc.VectorSubcoreMesh(core_axis_name="c", subcore_axis_name="s")
    f = pl.kernel(
        _gather_body,
        mesh=mesh,
        compiler_params=pltpu.CompilerParams(needs_layout_passes=False),
        out_type=jax.ShapeDtypeStruct((2 * D, B), jnp.float32),
        scratch_types=[
            pltpu.VMEM((_BPW,), jnp.int32),
            pltpu.VMEM((_BPW,), jnp.int32),
            pltpu.VMEM((_BPW,), jnp.int32),
            pltpu.VMEM((_BPW,), jnp.int32),
            pltpu.VMEM((_BPW,), jnp.int32),
            pltpu.VMEM((_BPW,), jnp.int32),
            pltpu.VMEM((_CH, 4 * D), jnp.float32),
            pltpu.VMEM((_CH, 4 * D), jnp.float32),
            pltpu.VMEM((2 * D, _BPW), jnp.float32),
            pltpu.SemaphoreType.DMA,
        ],
    )
    return f(uid, iid, upk, ipk)


_BN = 2048


def _mlp_body(x_ref, w1t_ref, b1_ref, w2t_ref, b2_ref, w3t_ref, b3_ref,
              o_ref):
    h = jnp.dot(w1t_ref[...], x_ref[...], preferred_element_type=jnp.float32)
    h = jnp.maximum(h + b1_ref[...], 0.0)
    h = jnp.dot(w2t_ref[...], h, preferred_element_type=jnp.float32)
    h = jnp.maximum(h + b2_ref[...], 0.0)
    o_ref[...] = jnp.dot(w3t_ref[...], h,
                         preferred_element_type=jnp.float32) + b3_ref[...]


def _tc_mlp(x, W1t, b1c, W2t, b2c, W3t, b3c):
    return pl.pallas_call(
        _mlp_body,
        grid=(B // _BN,),
        in_specs=[
            pl.BlockSpec((2 * D, _BN), lambda i: (0, i)),
            pl.BlockSpec((H1, 2 * D), lambda i: (0, 0)),
            pl.BlockSpec((H1, 1), lambda i: (0, 0)),
            pl.BlockSpec((H2, H1), lambda i: (0, 0)),
            pl.BlockSpec((H2, 1), lambda i: (0, 0)),
            pl.BlockSpec((1, H2), lambda i: (0, 0)),
            pl.BlockSpec((1, 1), lambda i: (0, 0)),
        ],
        out_specs=pl.BlockSpec((1, _BN), lambda i: (0, i)),
        out_shape=jax.ShapeDtypeStruct((1, B), jnp.float32),
    )(x, W1t, b1c, W2t, b2c, W3t, b3c)


def kernel(user_id, item_name, user_table, item_table, W1, b1, W2, b2, W3, b3):
    eye = jnp.eye(D, dtype=jnp.float32)
    upk, ipk = _pack_tables(user_table.T, item_table.T, eye)
    x = _sc_gather(user_id, item_name, upk, ipk)
    out_row = _tc_mlp(x, W1.T, b1.reshape(H1, 1), W2.T, b2.reshape(H2, 1),
                      W3.reshape(1, H2), b3.reshape(1, 1))
    return out_row.reshape(B, 1)


# speedup vs baseline: 3.0349x; 1.8399x over previous
import jax
import jax.numpy as jnp
from jax import lax
from jax.experimental import pallas as pl
from jax.experimental.pallas import tpu as pltpu
from jax.experimental.pallas import tpu_sc as plsc

B = 16384
D = 32
H1 = 256
H2 = 64
V = 1000001

_NC = 2
_NS = 16
_NW = _NC * _NS
_BPW = B // _NW
_CH = 128
_NCH = _BPW // _CH

_VBLK = 32768
_PBLK = _VBLK // 4
_NBLK = (V + _VBLK - 1) // _VBLK
_PROWS = _NBLK * _PBLK


def _pack_body(t_ref, p_ref):
    p_ref[...] = jnp.concatenate(
        [jnp.transpose(
            t_ref[:, j * _PBLK:(j + 1) * _PBLK].astype(jnp.bfloat16)
         ).astype(jnp.float32)
         for j in range(4)], axis=1)


def _pack_table(tabT):
    return pl.pallas_call(
        _pack_body,
        grid=(_NBLK,),
        in_specs=[pl.BlockSpec((D, _VBLK), lambda i: (0, i))],
        out_specs=pl.BlockSpec((_PBLK, 4 * D), lambda i: (i, 0)),
        out_shape=jax.ShapeDtypeStruct((_PROWS, 4 * D), jnp.float32),
    )(tabT)


def _gather_body(id_ref, pk_ref, out_ref,
                 idx_v, pr_v, ja_v, slot_a, slot_b, slot_c, slot_d,
                 xbuf_v, sem):
    wid = lax.axis_index("s") * _NC + lax.axis_index("c")
    base = wid * _BPW
    pltpu.sync_copy(id_ref.at[pl.ds(base, _BPW)], idx_v)

    for g in range(_BPW // 16):
        rv = idx_v[pl.ds(16 * g, 16)]
        pr_v[pl.ds(16 * g, 16)] = ((rv >> 15) << 13) | (rv & 8191)
        ja_v[pl.ds(16 * g, 16)] = ((rv >> 13) & 3) * D

    slots = (slot_a, slot_b, slot_c, slot_d)
    nbuf = len(slots)

    def fire(ch, slot):
        return pltpu.async_copy(
            pk_ref.at[pr_v.at[pl.ds(ch * _CH, _CH)]], slot, sem)

    def extract(ch, slot):
        def per_f(f, _):
            for g2 in range(_CH // 16):
                e_vec = lax.iota(jnp.int32, 16) + (16 * g2)
                ja = ja_v[pl.ds(ch * _CH + 16 * g2, 16)]
                vals = plsc.load_gather(slot, [e_vec, ja + f])
                xbuf_v[f, pl.ds(ch * _CH + 16 * g2, 16)] = vals
            return 0

        lax.fori_loop(0, D, per_f, 0)

    copies = {ch: fire(ch, slots[ch]) for ch in range(min(nbuf, _NCH))}
    for ch in range(_NCH):
        copies[ch].wait()
        extract(ch, slots[ch % nbuf])
        if ch + nbuf < _NCH:
            copies[ch + nbuf] = fire(ch + nbuf, slots[ch % nbuf])

    pltpu.sync_copy(xbuf_v, out_ref.at[:, pl.ds(base, _BPW)])


def _sc_gather(ids, pk):
    mesh = plsc.VectorSubcoreMesh(core_axis_name="c", subcore_axis_name="s")
    f = pl.kernel(
        _gather_body,
        mesh=mesh,
        compiler_params=pltpu.CompilerParams(needs_layout_passes=False),
        out_type=jax.ShapeDtypeStruct((D, B), jnp.float32),
        scratch_types=[
            pltpu.VMEM((_BPW,), jnp.int32),
            pltpu.VMEM((_BPW,), jnp.int32),
            pltpu.VMEM((_BPW,), jnp.int32),
            pltpu.VMEM((_CH, 4 * D), jnp.float32),
            pltpu.VMEM((_CH, 4 * D), jnp.float32),
            pltpu.VMEM((_CH, 4 * D), jnp.float32),
            pltpu.VMEM((_CH, 4 * D), jnp.float32),
            pltpu.VMEM((D, _BPW), jnp.float32),
            pltpu.SemaphoreType.DMA,
        ],
    )
    return f(ids, pk)


_BN = 2048


def _mlp_body(xu_ref, xi_ref, w1ut_ref, w1it_ref, b1_ref, w2t_ref, b2_ref,
              w3t_ref, b3_ref, o_ref):
    h = jnp.dot(w1ut_ref[...], xu_ref[...],
                preferred_element_type=jnp.float32)
    h = h + jnp.dot(w1it_ref[...], xi_ref[...],
                    preferred_element_type=jnp.float32)
    h = jnp.maximum(h + b1_ref[...], 0.0)
    h = jnp.dot(w2t_ref[...], h, preferred_element_type=jnp.float32)
    h = jnp.maximum(h + b2_ref[...], 0.0)
    o_ref[...] = jnp.dot(w3t_ref[...], h,
                         preferred_element_type=jnp.float32) + b3_ref[...]


def _tc_mlp(xu, xi, W1ut, W1it, b1c, W2t, b2c, W3t, b3c):
    return pl.pallas_call(
        _mlp_body,
        grid=(B // _BN,),
        in_specs=[
            pl.BlockSpec((D, _BN), lambda i: (0, i)),
            pl.BlockSpec((D, _BN), lambda i: (0, i)),
            pl.BlockSpec((H1, D), lambda i: (0, 0)),
            pl.BlockSpec((H1, D), lambda i: (0, 0)),
            pl.BlockSpec((H1, 1), lambda i: (0, 0)),
            pl.BlockSpec((H2, H1), lambda i: (0, 0)),
            pl.BlockSpec((H2, 1), lambda i: (0, 0)),
            pl.BlockSpec((1, H2), lambda i: (0, 0)),
            pl.BlockSpec((1, 1), lambda i: (0, 0)),
        ],
        out_specs=pl.BlockSpec((1, _BN), lambda i: (0, i)),
        out_shape=jax.ShapeDtypeStruct((1, B), jnp.float32),
    )(xu, xi, W1ut, W1it, b1c, W2t, b2c, W3t, b3c)


def kernel(user_id, item_name, user_table, item_table, W1, b1, W2, b2, W3, b3):
    upk = _pack_table(user_table.T)
    ipk = _pack_table(item_table.T)
    xu = _sc_gather(user_id, upk)
    xi = _sc_gather(item_name, ipk)
    out_row = _tc_mlp(xu, xi, W1[:D].T, W1[D:].T, b1.reshape(H1, 1),
                      W2.T, b2.reshape(H2, 1), W3.reshape(1, H2),
                      b3.reshape(1, 1))
    return out_row.reshape(B, 1)
